# baseline (speedup 1.0000x reference)
import functools

import jax
import jax.numpy as jnp
from jax import lax
from jax.experimental import pallas as pl
from jax.experimental.pallas import tpu as pltpu
from jax.experimental.pallas import tpu_sc as plsc

N = 10000
F = 128
H = 64
B = 64
OUT = 3
EPS = 1e-3

R = 200
G = N // R

NTILES = 16
CHUNK = 640
TAIL = N - (NTILES - 1) * CHUNK
NCOMP = 8


def _ln(h, gamma, beta):
    mu = jnp.mean(h, axis=-1, keepdims=True)
    var = jnp.mean((h - mu) * (h - mu), axis=-1, keepdims=True)
    return (h - mu) * lax.rsqrt(var + EPS) * gamma + beta


def _gcn_body(x_ref, a_ref, W1_ref, b1_ref, g1_ref, be1_ref, W2_ref, b2_ref,
              g2_ref, be2_ref, Wf_ref, bf_ref, Wa_ref, ba_ref, Woh_ref,
              Woq_ref, y_ref, u_scr, h1_scr):
    p = pl.program_id(0)
    r = pl.program_id(1)

    @pl.when((p == 0) & (r == 0))
    def _():
        u_scr[...] = jnp.dot(x_ref[...], W1_ref[...],
                             preferred_element_type=jnp.float32)

    @pl.when((p == 1) & (r == 0))
    def _():
        u_scr[...] = jnp.dot(h1_scr[...], W2_ref[...],
                             preferred_element_type=jnp.float32)

    acc = jnp.dot(a_ref[...], u_scr[...], preferred_element_type=jnp.float32)

    @pl.when(p == 0)
    def _():
        h = jnp.maximum(acc + b1_ref[...], 0.0)
        h1_scr[pl.ds(r * R, R), :] = _ln(h, g1_ref[...], be1_ref[...])
        y_ref[...] = jnp.zeros((R, NCOMP), jnp.float32)

    @pl.when(p == 1)
    def _():
        h2 = jnp.maximum(acc + b2_ref[...], 0.0)
        h = _ln(h2, g2_ref[...], be2_ref[...]) + h1_scr[pl.ds(r * R, R), :]
        feat = (jnp.dot(h, Wf_ref[...], preferred_element_type=jnp.float32)
                + bf_ref[...])
        ta = (jnp.dot(h, Wa_ref[...], preferred_element_type=jnp.float32)
              + ba_ref[...])
        attn = 1.0 / (1.0 + jnp.exp(-ta))
        fa = feat * attn
        pcol = jnp.dot(fa, Woh_ref[...], preferred_element_type=jnp.float32)
        q = jnp.dot(x_ref[pl.ds(r * R, R), F - 3:F], Woq_ref[...],
                    preferred_element_type=jnp.float32)
        e = 1.0 + jnp.maximum(x_ref[pl.ds(r * R, R), 0:1], 0.0)
        ecol = jnp.where(lax.broadcasted_iota(jnp.int32, (1, NCOMP), 1) == 6,
                         1.0, 0.0)
        y_ref[...] = pcol + e * q + e * ecol


def _seg_body(y_hbm, seg_hbm, bo_hbm, out_hbm,
              ychunk, segchunk, acc, tmp, acctot, bo_v, out_v, shared):
    cid = lax.axis_index("c")
    sid = lax.axis_index("s")
    lanes = lax.iota(jnp.int32, 16)

    @pl.when(cid == 0)
    def _():
        @pl.when(sid < NTILES - 1)
        def _():
            pltpu.sync_copy(y_hbm.at[pl.ds(sid * CHUNK * NCOMP, CHUNK * NCOMP)],
                            ychunk.at[pl.ds(0, CHUNK * NCOMP)])
            pltpu.sync_copy(seg_hbm.at[pl.ds(sid * CHUNK, CHUNK)],
                            segchunk.at[pl.ds(0, CHUNK)])

        @pl.when(sid == NTILES - 1)
        def _():
            pltpu.sync_copy(y_hbm.at[pl.ds((NTILES - 1) * CHUNK * NCOMP,
                                           TAIL * NCOMP)],
                            ychunk.at[pl.ds(0, TAIL * NCOMP)])
            pltpu.sync_copy(seg_hbm.at[pl.ds((NTILES - 1) * CHUNK, TAIL)],
                            segchunk.at[pl.ds(0, TAIL)])

        for k in range(B * NCOMP // 16):
            acc[pl.ds(k * 16, 16)] = jnp.zeros((16,), jnp.float32)

        rowsel = lax.shift_right_logical(lanes, 3)
        csel = jnp.bitwise_and(lanes, 7)

        def body(s, carry):
            for u in range(8):
                rb = s * 16 + u * 2
                vals = ychunk[pl.ds(s * 128 + u * 16, 16)]
                segs = plsc.load_gather(segchunk, [rowsel + rb])
                plsc.addupdate_scatter(acc, [segs * NCOMP + csel], vals)
            return carry

        nit = jnp.where(sid == NTILES - 1, TAIL // 16, CHUNK // 16)
        lax.fori_loop(0, nit, body, 0)
        pltpu.sync_copy(acc, shared.at[sid])

    plsc.subcore_barrier()

    @pl.when((cid == 0) & (sid == 0))
    def _():
        nv = B * NCOMP // 16
        for k in range(nv):
            acctot[pl.ds(k * 16, 16)] = jnp.zeros((16,), jnp.float32)
        for t in range(NTILES):
            pltpu.sync_copy(shared.at[t], tmp)
            for k in range(nv):
                acctot[pl.ds(k * 16, 16)] = (acctot[pl.ds(k * 16, 16)]
                                             + tmp[pl.ds(k * 16, 16)])
        pltpu.sync_copy(bo_hbm, bo_v)
        for c in range(OUT):
            boc = bo_v[pl.ds(c * 16, 16)]
            for j in range(B // 16):
                b_idx = (j * 16 + lax.iota(jnp.int32, 16)) * NCOMP
                P = plsc.load_gather(acctot, [b_idx + c])
                V = plsc.load_gather(acctot, [b_idx + (c + 3)])
                E = plsc.load_gather(acctot, [b_idx + 6])
                out_v[pl.ds(c * B + j * 16, 16)] = (
                    P + V / jnp.maximum(E, 1e-30) + boc)
        pltpu.sync_copy(out_v, out_hbm)


_seg_kernel = functools.partial(
    pl.kernel,
    out_type=jax.ShapeDtypeStruct((OUT * B,), jnp.float32),
    mesh=plsc.VectorSubcoreMesh(core_axis_name="c", subcore_axis_name="s"),
    compiler_params=pltpu.CompilerParams(needs_layout_passes=False),
    scratch_types=[
        pltpu.VMEM((CHUNK * NCOMP,), jnp.float32),
        pltpu.VMEM((CHUNK,), jnp.int32),
        pltpu.VMEM((B * NCOMP,), jnp.float32),
        pltpu.VMEM((B * NCOMP,), jnp.float32),
        pltpu.VMEM((B * NCOMP,), jnp.float32),
        pltpu.VMEM((OUT * 16,), jnp.float32),
        pltpu.VMEM((OUT * B,), jnp.float32),
        pltpu.VMEM_SHARED((NTILES, B * NCOMP), jnp.float32),
    ],
)(_seg_body)


def kernel(x, a, i, W1, b1, W2, b2, g1, be1, g2, be2, Wf, bf, Wa, ba, Wo, bo):
    b1r = b1.reshape(1, H)
    g1r = g1.reshape(1, H)
    be1r = be1.reshape(1, H)
    b2r = b2.reshape(1, H)
    g2r = g2.reshape(1, H)
    be2r = be2.reshape(1, H)
    bfr = bf.reshape(1, H)
    bar = ba.reshape(1, H)
    Woh = jnp.zeros((H, NCOMP), jnp.float32).at[:, :OUT].set(Wo[:H])
    Woq = jnp.zeros((OUT, NCOMP), jnp.float32).at[:, OUT:2 * OUT].set(Wo[H:])

    y = pl.pallas_call(
        _gcn_body,
        grid=(2, G),
        in_specs=[
            pl.BlockSpec((N, F), lambda p, r: (0, 0)),
            pl.BlockSpec((R, N), lambda p, r: (r, 0)),
            pl.BlockSpec((F, H), lambda p, r: (0, 0)),
            pl.BlockSpec((1, H), lambda p, r: (0, 0)),
            pl.BlockSpec((1, H), lambda p, r: (0, 0)),
            pl.BlockSpec((1, H), lambda p, r: (0, 0)),
            pl.BlockSpec((H, H), lambda p, r: (0, 0)),
            pl.BlockSpec((1, H), lambda p, r: (0, 0)),
            pl.BlockSpec((1, H), lambda p, r: (0, 0)),
            pl.BlockSpec((1, H), lambda p, r: (0, 0)),
            pl.BlockSpec((H, H), lambda p, r: (0, 0)),
            pl.BlockSpec((1, H), lambda p, r: (0, 0)),
            pl.BlockSpec((H, H), lambda p, r: (0, 0)),
            pl.BlockSpec((1, H), lambda p, r: (0, 0)),
            pl.BlockSpec((H, NCOMP), lambda p, r: (0, 0)),
            pl.BlockSpec((OUT, NCOMP), lambda p, r: (0, 0)),
        ],
        out_specs=pl.BlockSpec((R, NCOMP), lambda p, r: (r, 0)),
        out_shape=jax.ShapeDtypeStruct((N, NCOMP), jnp.float32),
        scratch_shapes=[pltpu.VMEM((N, H), jnp.float32),
                        pltpu.VMEM((N, H), jnp.float32)],
    )(x, a, W1, b1r, g1r, be1r, W2, b2r, g2r, be2r, Wf, bfr, Wa, bar, Woh, Woq)

    seg = i.astype(jnp.int32)
    bo48 = jnp.repeat(bo, 16)

    out_flat = _seg_kernel(y.reshape(-1), seg, bo48)
    return out_flat.reshape(OUT, B).T


# speedup vs baseline: 1.0523x; 1.0523x over previous
import functools

import jax
import jax.numpy as jnp
from jax import lax
from jax.experimental import pallas as pl
from jax.experimental.pallas import tpu as pltpu
from jax.experimental.pallas import tpu_sc as plsc

N = 10000
F = 128
H = 64
B = 64
OUT = 3
EPS = 1e-3

R = 400
G = N // R

NTILES = 16
CHUNK = 640
TAIL = N - (NTILES - 1) * CHUNK
NCOMP = 8


def _ln(h, gamma, beta):
    mu = jnp.mean(h, axis=-1, keepdims=True)
    var = jnp.mean((h - mu) * (h - mu), axis=-1, keepdims=True)
    return (h - mu) * lax.rsqrt(var + EPS) * gamma + beta


def _gcn_body(x_ref, a_ref, W1_ref, b1_ref, g1_ref, be1_ref, W2_ref, b2_ref,
              g2_ref, be2_ref, Wf_ref, bf_ref, Wa_ref, ba_ref, Woh_ref,
              Woq_ref, y_ref, u_scr, h1_scr):
    p = pl.program_id(0)
    r = pl.program_id(1)

    @pl.when((p == 0) & (r == 0))
    def _():
        u_scr[...] = jnp.dot(x_ref[...], W1_ref[...],
                             preferred_element_type=jnp.float32)

    @pl.when((p == 1) & (r == 0))
    def _():
        u_scr[...] = jnp.dot(h1_scr[...], W2_ref[...],
                             preferred_element_type=jnp.float32)

    acc = jnp.dot(a_ref[...], u_scr[...], preferred_element_type=jnp.float32)

    @pl.when(p == 0)
    def _():
        h = jnp.maximum(acc + b1_ref[...], 0.0)
        h1_scr[pl.ds(r * R, R), :] = _ln(h, g1_ref[...], be1_ref[...])
        y_ref[...] = jnp.zeros((R, NCOMP), jnp.float32)

    @pl.when(p == 1)
    def _():
        h2 = jnp.maximum(acc + b2_ref[...], 0.0)
        h = _ln(h2, g2_ref[...], be2_ref[...]) + h1_scr[pl.ds(r * R, R), :]
        feat = (jnp.dot(h, Wf_ref[...], preferred_element_type=jnp.float32)
                + bf_ref[...])
        ta = (jnp.dot(h, Wa_ref[...], preferred_element_type=jnp.float32)
              + ba_ref[...])
        attn = 1.0 / (1.0 + jnp.exp(-ta))
        fa = feat * attn
        pcol = jnp.dot(fa, Woh_ref[...], preferred_element_type=jnp.float32)
        q = jnp.dot(x_ref[pl.ds(r * R, R), F - 3:F], Woq_ref[...],
                    preferred_element_type=jnp.float32)
        e = 1.0 + jnp.maximum(x_ref[pl.ds(r * R, R), 0:1], 0.0)
        ecol = jnp.where(lax.broadcasted_iota(jnp.int32, (1, NCOMP), 1) == 6,
                         1.0, 0.0)
        y_ref[...] = pcol + e * q + e * ecol


def _seg_body(y_hbm, seg_hbm, bo_hbm, out_hbm,
              ychunk, segchunk, acc, tmp, acctot, bo_v, out_v, shared):
    cid = lax.axis_index("c")
    sid = lax.axis_index("s")
    lanes = lax.iota(jnp.int32, 16)

    @pl.when(cid == 0)
    def _():
        @pl.when(sid < NTILES - 1)
        def _():
            pltpu.sync_copy(y_hbm.at[pl.ds(sid * CHUNK * NCOMP, CHUNK * NCOMP)],
                            ychunk.at[pl.ds(0, CHUNK * NCOMP)])
            pltpu.sync_copy(seg_hbm.at[pl.ds(sid * CHUNK, CHUNK)],
                            segchunk.at[pl.ds(0, CHUNK)])

        @pl.when(sid == NTILES - 1)
        def _():
            pltpu.sync_copy(y_hbm.at[pl.ds((NTILES - 1) * CHUNK * NCOMP,
                                           TAIL * NCOMP)],
                            ychunk.at[pl.ds(0, TAIL * NCOMP)])
            pltpu.sync_copy(seg_hbm.at[pl.ds((NTILES - 1) * CHUNK, TAIL)],
                            segchunk.at[pl.ds(0, TAIL)])

        for k in range(B * NCOMP // 16):
            acc[pl.ds(k * 16, 16)] = jnp.zeros((16,), jnp.float32)

        rowsel = lax.shift_right_logical(lanes, 3)
        csel = jnp.bitwise_and(lanes, 7)

        def body(s, carry):
            for u in range(8):
                rb = s * 16 + u * 2
                vals = ychunk[pl.ds(s * 128 + u * 16, 16)]
                segs = plsc.load_gather(segchunk, [rowsel + rb])
                plsc.addupdate_scatter(acc, [segs * NCOMP + csel], vals)
            return carry

        nit = jnp.where(sid == NTILES - 1, TAIL // 16, CHUNK // 16)
        lax.fori_loop(0, nit, body, 0)
        pltpu.sync_copy(acc, shared.at[sid])

    plsc.subcore_barrier()

    @pl.when((cid == 0) & (sid == 0))
    def _():
        nv = B * NCOMP // 16
        for k in range(nv):
            acctot[pl.ds(k * 16, 16)] = jnp.zeros((16,), jnp.float32)
        for t in range(NTILES):
            pltpu.sync_copy(shared.at[t], tmp)
            for k in range(nv):
                acctot[pl.ds(k * 16, 16)] = (acctot[pl.ds(k * 16, 16)]
                                             + tmp[pl.ds(k * 16, 16)])
        pltpu.sync_copy(bo_hbm, bo_v)
        for c in range(OUT):
            boc = bo_v[pl.ds(c * 16, 16)]
            for j in range(B // 16):
                b_idx = (j * 16 + lax.iota(jnp.int32, 16)) * NCOMP
                P = plsc.load_gather(acctot, [b_idx + c])
                V = plsc.load_gather(acctot, [b_idx + (c + 3)])
                E = plsc.load_gather(acctot, [b_idx + 6])
                out_v[pl.ds(c * B + j * 16, 16)] = (
                    P + V / jnp.maximum(E, 1e-30) + boc)
        pltpu.sync_copy(out_v, out_hbm)


_seg_kernel = functools.partial(
    pl.kernel,
    out_type=jax.ShapeDtypeStruct((OUT * B,), jnp.float32),
    mesh=plsc.VectorSubcoreMesh(core_axis_name="c", subcore_axis_name="s"),
    compiler_params=pltpu.CompilerParams(needs_layout_passes=False),
    scratch_types=[
        pltpu.VMEM((CHUNK * NCOMP,), jnp.float32),
        pltpu.VMEM((CHUNK,), jnp.int32),
        pltpu.VMEM((B * NCOMP,), jnp.float32),
        pltpu.VMEM((B * NCOMP,), jnp.float32),
        pltpu.VMEM((B * NCOMP,), jnp.float32),
        pltpu.VMEM((OUT * 16,), jnp.float32),
        pltpu.VMEM((OUT * B,), jnp.float32),
        pltpu.VMEM_SHARED((NTILES, B * NCOMP), jnp.float32),
    ],
)(_seg_body)


def kernel(x, a, i, W1, b1, W2, b2, g1, be1, g2, be2, Wf, bf, Wa, ba, Wo, bo):
    b1r = b1.reshape(1, H)
    g1r = g1.reshape(1, H)
    be1r = be1.reshape(1, H)
    b2r = b2.reshape(1, H)
    g2r = g2.reshape(1, H)
    be2r = be2.reshape(1, H)
    bfr = bf.reshape(1, H)
    bar = ba.reshape(1, H)
    Woh = jnp.zeros((H, NCOMP), jnp.float32).at[:, :OUT].set(Wo[:H])
    Woq = jnp.zeros((OUT, NCOMP), jnp.float32).at[:, OUT:2 * OUT].set(Wo[H:])

    y = pl.pallas_call(
        _gcn_body,
        grid=(2, G),
        in_specs=[
            pl.BlockSpec((N, F), lambda p, r: (0, 0)),
            pl.BlockSpec((R, N), lambda p, r: (r, 0)),
            pl.BlockSpec((F, H), lambda p, r: (0, 0)),
            pl.BlockSpec((1, H), lambda p, r: (0, 0)),
            pl.BlockSpec((1, H), lambda p, r: (0, 0)),
            pl.BlockSpec((1, H), lambda p, r: (0, 0)),
            pl.BlockSpec((H, H), lambda p, r: (0, 0)),
            pl.BlockSpec((1, H), lambda p, r: (0, 0)),
            pl.BlockSpec((1, H), lambda p, r: (0, 0)),
            pl.BlockSpec((1, H), lambda p, r: (0, 0)),
            pl.BlockSpec((H, H), lambda p, r: (0, 0)),
            pl.BlockSpec((1, H), lambda p, r: (0, 0)),
            pl.BlockSpec((H, H), lambda p, r: (0, 0)),
            pl.BlockSpec((1, H), lambda p, r: (0, 0)),
            pl.BlockSpec((H, NCOMP), lambda p, r: (0, 0)),
            pl.BlockSpec((OUT, NCOMP), lambda p, r: (0, 0)),
        ],
        out_specs=pl.BlockSpec((R, NCOMP), lambda p, r: (r, 0)),
        out_shape=jax.ShapeDtypeStruct((N, NCOMP), jnp.float32),
        scratch_shapes=[pltpu.VMEM((N, H), jnp.float32),
                        pltpu.VMEM((N, H), jnp.float32)],
    )(x, a, W1, b1r, g1r, be1r, W2, b2r, g2r, be2r, Wf, bfr, Wa, bar, Woh, Woq)

    seg = i.astype(jnp.int32)
    bo48 = jnp.repeat(bo, 16)

    out_flat = _seg_kernel(y.reshape(-1), seg, bo48)
    return out_flat.reshape(OUT, B).T


# speedup vs baseline: 1.0611x; 1.0083x over previous
import functools

import jax
import jax.numpy as jnp
from jax import lax
from jax.experimental import pallas as pl
from jax.experimental.pallas import tpu as pltpu
from jax.experimental.pallas import tpu_sc as plsc

N = 10000
F = 128
H = 64
B = 64
OUT = 3
EPS = 1e-3

R = 400
G = N // R

NTILES = 16
CHUNK = 640
TAIL = N - (NTILES - 1) * CHUNK
NCOMP = 8


def _ln(h, gamma, beta):
    mu = jnp.mean(h, axis=-1, keepdims=True)
    var = jnp.mean((h - mu) * (h - mu), axis=-1, keepdims=True)
    return (h - mu) * (1.0 / jnp.sqrt(var + EPS)) * gamma + beta


def _gcn_body(x_ref, a_ref, W1_ref, b1_ref, g1_ref, be1_ref, W2_ref, b2_ref,
              g2_ref, be2_ref, Wf_ref, bf_ref, Wa_ref, ba_ref, Woh_ref,
              Woq_ref, y_ref, u_scr, h1_scr):
    p = pl.program_id(0)
    r = pl.program_id(1)

    @pl.when((p == 0) & (r == 0))
    def _():
        u_scr[...] = jnp.dot(x_ref[...], W1_ref[...],
                             preferred_element_type=jnp.float32)

    @pl.when((p == 1) & (r == 0))
    def _():
        u_scr[...] = jnp.dot(h1_scr[...], W2_ref[...],
                             preferred_element_type=jnp.float32)

    acc = jnp.dot(a_ref[...], u_scr[...], preferred_element_type=jnp.float32)

    @pl.when(p == 0)
    def _():
        h = jnp.maximum(acc + b1_ref[...], 0.0)
        h1_scr[pl.ds(r * R, R), :] = _ln(h, g1_ref[...], be1_ref[...])
        y_ref[...] = jnp.zeros((R, NCOMP), jnp.float32)

    @pl.when(p == 1)
    def _():
        h2 = jnp.maximum(acc + b2_ref[...], 0.0)
        h = _ln(h2, g2_ref[...], be2_ref[...]) + h1_scr[pl.ds(r * R, R), :]
        feat = (jnp.dot(h, Wf_ref[...], preferred_element_type=jnp.float32)
                + bf_ref[...])
        ta = (jnp.dot(h, Wa_ref[...], preferred_element_type=jnp.float32)
              + ba_ref[...])
        attn = 1.0 / (1.0 + jnp.exp(-ta))
        fa = feat * attn
        pcol = jnp.dot(fa, Woh_ref[...], preferred_element_type=jnp.float32)
        q = jnp.dot(x_ref[pl.ds(r * R, R), F - 3:F], Woq_ref[...],
                    preferred_element_type=jnp.float32)
        e = 1.0 + jnp.maximum(x_ref[pl.ds(r * R, R), 0:1], 0.0)
        ecol = jnp.where(lax.broadcasted_iota(jnp.int32, (1, NCOMP), 1) == 6,
                         1.0, 0.0)
        y_ref[...] = pcol + e * q + e * ecol


def _seg_body(y_hbm, seg_hbm, bo_hbm, out_hbm,
              ychunk, segchunk, acc, tmp, acctot, bo_v, out_v, shared):
    cid = lax.axis_index("c")
    sid = lax.axis_index("s")
    lanes = lax.iota(jnp.int32, 16)

    @pl.when(cid == 0)
    def _():
        @pl.when(sid < NTILES - 1)
        def _():
            pltpu.sync_copy(y_hbm.at[pl.ds(sid * CHUNK * NCOMP, CHUNK * NCOMP)],
                            ychunk.at[pl.ds(0, CHUNK * NCOMP)])
            pltpu.sync_copy(seg_hbm.at[pl.ds(sid * CHUNK, CHUNK)],
                            segchunk.at[pl.ds(0, CHUNK)])

        @pl.when(sid == NTILES - 1)
        def _():
            pltpu.sync_copy(y_hbm.at[pl.ds((NTILES - 1) * CHUNK * NCOMP,
                                           TAIL * NCOMP)],
                            ychunk.at[pl.ds(0, TAIL * NCOMP)])
            pltpu.sync_copy(seg_hbm.at[pl.ds((NTILES - 1) * CHUNK, TAIL)],
                            segchunk.at[pl.ds(0, TAIL)])

        for k in range(B * NCOMP // 16):
            acc[pl.ds(k * 16, 16)] = jnp.zeros((16,), jnp.float32)

        rowsel = lax.shift_right_logical(lanes, 3)
        csel = jnp.bitwise_and(lanes, 7)

        def body(s, carry):
            for u in range(8):
                rb = s * 16 + u * 2
                vals = ychunk[pl.ds(s * 128 + u * 16, 16)]
                segs = plsc.load_gather(segchunk, [rowsel + rb])
                plsc.addupdate_scatter(acc, [segs * NCOMP + csel], vals)
            return carry

        nit = jnp.where(sid == NTILES - 1, TAIL // 16, CHUNK // 16)
        lax.fori_loop(0, nit, body, 0)
        pltpu.sync_copy(acc, shared.at[sid])

    plsc.subcore_barrier()

    @pl.when((cid == 0) & (sid == 0))
    def _():
        nv = B * NCOMP // 16
        for k in range(nv):
            acctot[pl.ds(k * 16, 16)] = jnp.zeros((16,), jnp.float32)
        for t in range(NTILES):
            pltpu.sync_copy(shared.at[t], tmp)
            for k in range(nv):
                acctot[pl.ds(k * 16, 16)] = (acctot[pl.ds(k * 16, 16)]
                                             + tmp[pl.ds(k * 16, 16)])
        pltpu.sync_copy(bo_hbm, bo_v)
        for c in range(OUT):
            boc = bo_v[pl.ds(c * 16, 16)]
            for j in range(B // 16):
                b_idx = (j * 16 + lax.iota(jnp.int32, 16)) * NCOMP
                P = plsc.load_gather(acctot, [b_idx + c])
                V = plsc.load_gather(acctot, [b_idx + (c + 3)])
                E = plsc.load_gather(acctot, [b_idx + 6])
                res = P + V / jnp.maximum(E, 1e-30) + boc
                plsc.store_scatter(
                    out_v, [(j * 16 + lax.iota(jnp.int32, 16)) * OUT + c], res)
        pltpu.sync_copy(out_v, out_hbm)


_seg_kernel = functools.partial(
    pl.kernel,
    out_type=jax.ShapeDtypeStruct((B * OUT,), jnp.float32),
    mesh=plsc.VectorSubcoreMesh(core_axis_name="c", subcore_axis_name="s"),
    compiler_params=pltpu.CompilerParams(needs_layout_passes=False),
    scratch_types=[
        pltpu.VMEM((CHUNK * NCOMP,), jnp.float32),
        pltpu.VMEM((CHUNK,), jnp.int32),
        pltpu.VMEM((B * NCOMP,), jnp.float32),
        pltpu.VMEM((B * NCOMP,), jnp.float32),
        pltpu.VMEM((B * NCOMP,), jnp.float32),
        pltpu.VMEM((OUT * 16,), jnp.float32),
        pltpu.VMEM((B * OUT,), jnp.float32),
        pltpu.VMEM_SHARED((NTILES, B * NCOMP), jnp.float32),
    ],
)(_seg_body)


def kernel(x, a, i, W1, b1, W2, b2, g1, be1, g2, be2, Wf, bf, Wa, ba, Wo, bo):
    b1r = b1.reshape(1, H)
    g1r = g1.reshape(1, H)
    be1r = be1.reshape(1, H)
    b2r = b2.reshape(1, H)
    g2r = g2.reshape(1, H)
    be2r = be2.reshape(1, H)
    bfr = bf.reshape(1, H)
    bar = ba.reshape(1, H)
    Woh = jnp.zeros((H, NCOMP), jnp.float32).at[:, :OUT].set(Wo[:H])
    Woq = jnp.zeros((OUT, NCOMP), jnp.float32).at[:, OUT:2 * OUT].set(Wo[H:])

    y = pl.pallas_call(
        _gcn_body,
        grid=(2, G),
        in_specs=[
            pl.BlockSpec((N, F), lambda p, r: (0, 0)),
            pl.BlockSpec((R, N), lambda p, r: (r, 0)),
            pl.BlockSpec((F, H), lambda p, r: (0, 0)),
            pl.BlockSpec((1, H), lambda p, r: (0, 0)),
            pl.BlockSpec((1, H), lambda p, r: (0, 0)),
            pl.BlockSpec((1, H), lambda p, r: (0, 0)),
            pl.BlockSpec((H, H), lambda p, r: (0, 0)),
            pl.BlockSpec((1, H), lambda p, r: (0, 0)),
            pl.BlockSpec((1, H), lambda p, r: (0, 0)),
            pl.BlockSpec((1, H), lambda p, r: (0, 0)),
            pl.BlockSpec((H, H), lambda p, r: (0, 0)),
            pl.BlockSpec((1, H), lambda p, r: (0, 0)),
            pl.BlockSpec((H, H), lambda p, r: (0, 0)),
            pl.BlockSpec((1, H), lambda p, r: (0, 0)),
            pl.BlockSpec((H, NCOMP), lambda p, r: (0, 0)),
            pl.BlockSpec((OUT, NCOMP), lambda p, r: (0, 0)),
        ],
        out_specs=pl.BlockSpec((R, NCOMP), lambda p, r: (r, 0)),
        out_shape=jax.ShapeDtypeStruct((N, NCOMP), jnp.float32),
        scratch_shapes=[pltpu.VMEM((N, H), jnp.float32),
                        pltpu.VMEM((N, H), jnp.float32)],
    )(x, a, W1, b1r, g1r, be1r, W2, b2r, g2r, be2r, Wf, bfr, Wa, bar, Woh, Woq)

    seg = i.astype(jnp.int32)
    bo48 = jnp.repeat(bo, 16)

    out_flat = _seg_kernel(y.reshape(-1), seg, bo48)
    return out_flat.reshape(B, OUT)
